# baseline (speedup 1.0000x reference)
import functools

import jax
import jax.numpy as jnp
from jax import lax
from jax.experimental import pallas as pl
from jax.experimental.pallas import tpu as pltpu
from jax.experimental.pallas import tpu_sc as plsc

TN = 512
SLAB_ROWS = 512


def _build_m_body(ei_ref, m_ref, deg_ref, invc_ref,
                  rowv, colv, lapv, majv, degv, disv, invcv, zbuf, z1d, ohbuf,
                  deg_sp, invc_sp, slab, sem):
    N = deg_ref.shape[0]
    E = ei_ref.shape[1]
    EPW = E // 16
    NV = EPW // 16
    SLAB_FLAT = SLAB_ROWS * N
    c = lax.axis_index("c")
    s = lax.axis_index("s")
    base = s * EPW

    with jax.named_scope("sc_load_edges"):
        pltpu.sync_copy(ei_ref.at[0, pl.ds(base, EPW)], rowv)
        pltpu.sync_copy(ei_ref.at[1, pl.ds(base, EPW)], colv)

        def zloop(i, _):
            zbuf[i] = jnp.zeros((16,), jnp.float32)
            return 0
        lax.fori_loop(0, zbuf.shape[0], zloop, 0)

        def z1loop(i, _):
            z1d[pl.ds(i * 16, 16)] = jnp.zeros((16,), jnp.float32)
            return 0
        lax.fori_loop(0, z1d.shape[0] // 16, z1loop, 0)

    with jax.named_scope("sc_deg"):
        pltpu.sync_copy(z1d, deg_sp.at[pl.ds(s * (N // 16), N // 16)])
        plsc.subcore_barrier()

        def ones_loop(i, _):
            r16 = rowv[pl.ds(i * 16, 16)]
            c16 = colv[pl.ds(i * 16, 16)]
            lapv[pl.ds(i * 16, 16)] = jnp.where(r16 != c16, 1.0, 0.0)
            return 0
        lax.fori_loop(0, NV, ones_loop, 0)
        pltpu.sync_copy(lapv, deg_sp.at[rowv], add=True)
        plsc.subcore_barrier()

    ns_dis = jax.named_scope("sc_dis")
    ns_dis.__enter__()
    pltpu.sync_copy(deg_sp, degv)
    plsc.subcore_barrier()
    NS = N // 16

    def dis_loop(i, _):
        off = s * NS + i * 16
        d16 = degv[pl.ds(off, 16)]
        pos = d16 > 0.0
        dsafe = jnp.where(pos, d16, 1.0)
        y = 1.0 / dsafe
        half = 0.5 * dsafe
        for _ in range(18):
            y = y * (1.5 - half * y * y)
        disv[pl.ds(off, 16)] = jnp.where(pos, y, 0.0)
        invcv[pl.ds(off, 16)] = 1.0 / (d16 + 2.0)
        return 0
    lax.fori_loop(0, NS // 16, dis_loop, 0)
    pltpu.sync_copy(disv.at[pl.ds(s * NS, NS)], deg_sp.at[pl.ds(s * NS, NS)])
    pltpu.sync_copy(invcv.at[pl.ds(s * NS, NS)], invc_sp.at[pl.ds(s * NS, NS)])
    plsc.subcore_barrier()
    pltpu.sync_copy(deg_sp, disv)

    @pl.when(jnp.logical_and(c == 0, s == 0))
    def _():
        pltpu.sync_copy(degv, deg_ref)
        pltpu.sync_copy(invc_sp, invcv)
        pltpu.sync_copy(invcv, invc_ref)
    ns_dis.__exit__(None, None, None)

    with jax.named_scope("sc_lap"):
        def lap_loop(i, _):
            r16 = rowv[pl.ds(i * 16, 16)]
            c16 = colv[pl.ds(i * 16, 16)]
            a = plsc.load_gather(disv, [r16])
            b = plsc.load_gather(disv, [c16])
            lapv[pl.ds(i * 16, 16)] = jnp.where(r16 != c16, -(a * b), 0.0)
            return 0
        lax.fori_loop(0, NV, lap_loop, 0)

    with jax.named_scope("sc_onehot"):
        def oh_loop(i, _):
            c16 = colv[pl.ds(i * 16, 16)]
            lap16 = lapv[pl.ds(i * 16, 16)]
            lane = lax.rem(c16, 16)
            eids = i * 16 + lax.iota(jnp.int32, 16)
            for l in range(16):
                coll = jnp.where(lane == l, lap16, 0.0)
                plsc.store_scatter(ohbuf, [eids, jnp.full((16,), l, jnp.int32)], coll)
            return 0
        lax.fori_loop(0, NV, oh_loop, 0)

    SLAB_MAJ = SLAB_FLAT // 16
    SROWS = SLAB_MAJ // 16
    ZROWS = zbuf.shape[0]
    for slab_i in range(2):
        slab_id = c * 2 + slab_i
        lo = slab_id * SLAB_ROWS
        lo_flat = lo * N

        with jax.named_scope(f"sc_zero{slab_i}"):
            nz = SROWS // ZROWS
            zcps = [pltpu.async_copy(zbuf, slab.at[pl.ds(s * SROWS + k * ZROWS, ZROWS)], sem)
                    for k in range(nz)]
            for cp in zcps:
                cp.wait()
            plsc.subcore_barrier()

        with jax.named_scope(f"sc_scatter{slab_i}"):
            def maj_loop(i, _):
                r16 = rowv[pl.ds(i * 16, 16)]
                c16 = colv[pl.ds(i * 16, 16)]
                maj = lax.div(r16 * N + c16 - lo_flat, 16)
                ok = jnp.logical_and(r16 >= lo, r16 < lo + SLAB_ROWS)
                majv[pl.ds(i * 16, 16)] = jnp.where(ok, maj, SLAB_MAJ)
                return 0
            lax.fori_loop(0, NV, maj_loop, 0)
            pltpu.sync_copy(ohbuf, slab.at[majv], add=True)
            plsc.subcore_barrier()

        with jax.named_scope(f"sc_wb{slab_i}"):
            off = pl.multiple_of((lo_flat // 16) + s * SROWS, 8)
            pltpu.sync_copy(slab.at[pl.ds(s * SROWS, SROWS)],
                            m_ref.at[pl.ds(off, SROWS)])


def _make_build_m(N, E):
    EPW = E // 16
    SLAB_MAJ = SLAB_ROWS * N // 16
    mesh = plsc.VectorSubcoreMesh(core_axis_name="c", subcore_axis_name="s")
    return pl.kernel(
        _build_m_body,
        out_type=[
            jax.ShapeDtypeStruct((N * N // 16, 16), jnp.float32),
            jax.ShapeDtypeStruct((N,), jnp.float32),
            jax.ShapeDtypeStruct((N,), jnp.float32),
        ],
        mesh=mesh,
        compiler_params=pltpu.CompilerParams(
            needs_layout_passes=False, use_tc_tiling_on_sc=False),
        scratch_types=[
            pltpu.VMEM((EPW,), jnp.int32),
            pltpu.VMEM((EPW,), jnp.int32),
            pltpu.VMEM((EPW,), jnp.float32),
            pltpu.VMEM((EPW,), jnp.int32),
            pltpu.VMEM((N,), jnp.float32),
            pltpu.VMEM((N,), jnp.float32),
            pltpu.VMEM((N,), jnp.float32),
            pltpu.VMEM((512, 16), jnp.float32),
            pltpu.VMEM((N // 16,), jnp.float32),
            pltpu.VMEM((EPW, 16), jnp.float32),
            pltpu.VMEM_SHARED((N,), jnp.float32),
            pltpu.VMEM_SHARED((N,), jnp.float32),
            pltpu.VMEM_SHARED((SLAB_MAJ + 8, 16), jnp.float32),
            pltpu.SemaphoreType.DMA,
        ],
    )


def _prep_body(sa_ref, x_ref, tax0_ref):
    TB = sa_ref.shape[1]
    rloc = lax.broadcasted_iota(jnp.int32, (TB, TB), 0)
    cloc = lax.broadcasted_iota(jnp.int32, (TB, TB), 1)
    d = jnp.sum(jnp.where(rloc == cloc, sa_ref[0], 0.0), axis=1)
    tax0_ref[0] = d[:, None] * x_ref[0]


def _tax1_body(m_ref, sa_ref, tax0_ref, invc_ref, tax1_ref):
    b = pl.program_id(1)
    N = m_ref.shape[1]
    a = m_ref[...] * sa_ref[0]
    tax0 = tax0_ref[0, pl.ds(b * N, N), :]
    acc = jnp.dot(a, tax0, preferred_element_type=jnp.float32)
    tax1_ref[0] = invc_ref[...] * acc


def _out_body(m_ref, tax1_ref, tax0_ref, invc_ref, w_ref, b_ref, out_ref):
    i = pl.program_id(0)
    b = pl.program_id(1)
    N = m_ref.shape[1]
    tax1_full = tax1_ref[0, pl.ds(b * N, N), :]
    s2 = jnp.dot(m_ref[...], tax1_full, preferred_element_type=jnp.float32)
    tax0_t = tax0_ref[0]
    tax2 = 2.0 * invc_ref[...] * s2 - tax0_t
    tax1_t = tax1_ref[0, pl.ds(b * N + i * TN, TN), :]
    out = jnp.dot(tax0_t, w_ref[0], preferred_element_type=jnp.float32)
    out = out + jnp.dot(tax1_t, w_ref[1], preferred_element_type=jnp.float32)
    out = out + jnp.dot(tax2, w_ref[2], preferred_element_type=jnp.float32)
    out_ref[0] = out + b_ref[...]


@jax.jit
def _run(x, edge_index, spatial_attention, weight, bias):
    B, N, F = x.shape
    E = edge_index.shape[1]

    m_2d, deg, invc1 = _make_build_m(N, E)(edge_index)
    M = m_2d.reshape(N, N)
    invc = invc1.reshape(N, 1)

    TB = 128
    tax0 = pl.pallas_call(
        _prep_body,
        grid=(B, N // TB),
        in_specs=[
            pl.BlockSpec((1, TB, TB), lambda b, i: (b, i, i)),
            pl.BlockSpec((1, TB, F), lambda b, i: (b, i, 0)),
        ],
        out_specs=pl.BlockSpec((1, TB, F), lambda b, i: (b, i, 0)),
        out_shape=jax.ShapeDtypeStruct((B, N, F), jnp.float32),
    )(spatial_attention, x)

    T = N // TN
    grid = (T, B)
    tax0_flat = tax0.reshape(1, B * N, F)
    tax1 = pl.pallas_call(
        _tax1_body,
        grid=grid,
        in_specs=[
            pl.BlockSpec((TN, N), lambda i, b: (i, 0)),
            pl.BlockSpec((1, TN, N), lambda i, b: (b, i, 0)),
            pl.BlockSpec((1, B * N, F), lambda i, b: (0, 0, 0)),
            pl.BlockSpec((TN, 1), lambda i, b: (i, 0)),
        ],
        out_specs=pl.BlockSpec((1, TN, F), lambda i, b: (b, i, 0)),
        out_shape=jax.ShapeDtypeStruct((B, N, F), jnp.float32),
    )(M, spatial_attention, tax0_flat, invc)

    out = pl.pallas_call(
        _out_body,
        grid=grid,
        in_specs=[
            pl.BlockSpec((TN, N), lambda i, b: (i, 0)),
            pl.BlockSpec((1, B * N, F), lambda i, b: (0, 0, 0)),
            pl.BlockSpec((1, TN, F), lambda i, b: (b, i, 0)),
            pl.BlockSpec((TN, 1), lambda i, b: (i, 0)),
            pl.BlockSpec((3, F, F), lambda i, b: (0, 0, 0)),
            pl.BlockSpec((1, F), lambda i, b: (0, 0)),
        ],
        out_specs=pl.BlockSpec((1, TN, F), lambda i, b: (b, i, 0)),
        out_shape=jax.ShapeDtypeStruct((B, N, F), jnp.float32),
    )(M, tax1.reshape(1, B * N, F), tax0, invc, weight, bias.reshape(1, F))
    return out


def kernel(x, edge_index, spatial_attention, weight, bias):
    return _run(x, edge_index, spatial_attention, weight, bias)


# speedup vs baseline: 1.3441x; 1.3441x over previous
import functools

import jax
import jax.numpy as jnp
from jax import lax
from jax.experimental import pallas as pl
from jax.experimental.pallas import tpu as pltpu
from jax.experimental.pallas import tpu_sc as plsc

TN = 512
SLAB_ROWS = 512


def _build_m_body(ei_ref, m_ref, deg_ref, invc_ref,
                  rowv, colv, lapv, cmaj, clap, cmaj2, degv, disv, invcv,
                  zbuf, z1d, deg_sp, invc_sp, slab, sem):
    N = deg_ref.shape[0]
    E = ei_ref.shape[1]
    EPW = E // 16
    NV = EPW // 16
    SLAB_FLAT = SLAB_ROWS * N
    c = lax.axis_index("c")
    s = lax.axis_index("s")
    base = s * EPW

    with jax.named_scope("sc_load_edges"):
        pltpu.sync_copy(ei_ref.at[0, pl.ds(base, EPW)], rowv)
        pltpu.sync_copy(ei_ref.at[1, pl.ds(base, EPW)], colv)

        def zloop(i, _):
            zbuf[pl.ds(i * 16, 16)] = jnp.zeros((16,), jnp.float32)
            return 0
        lax.fori_loop(0, zbuf.shape[0] // 16, zloop, 0)

        def z1loop(i, _):
            z1d[pl.ds(i * 16, 16)] = jnp.zeros((16,), jnp.float32)
            return 0
        lax.fori_loop(0, z1d.shape[0] // 16, z1loop, 0)

    with jax.named_scope("sc_deg"):
        pltpu.sync_copy(z1d, deg_sp.at[pl.ds(s * (N // 16), N // 16)])
        plsc.subcore_barrier()

        def ones_loop(i, _):
            r16 = rowv[pl.ds(i * 16, 16)]
            c16 = colv[pl.ds(i * 16, 16)]
            lapv[pl.ds(i * 16, 16)] = jnp.where(r16 != c16, 1.0, 0.0)
            return 0
        lax.fori_loop(0, NV, ones_loop, 0)
        pltpu.sync_copy(lapv, deg_sp.at[rowv], add=True)
        plsc.subcore_barrier()

    ns_dis = jax.named_scope("sc_dis")
    ns_dis.__enter__()
    pltpu.sync_copy(deg_sp, degv)
    plsc.subcore_barrier()
    NS = N // 16

    def dis_loop(i, _):
        off = s * NS + i * 16
        d16 = degv[pl.ds(off, 16)]
        pos = d16 > 0.0
        dsafe = jnp.where(pos, d16, 1.0)
        y = 1.0 / dsafe
        half = 0.5 * dsafe
        for _ in range(18):
            y = y * (1.5 - half * y * y)
        disv[pl.ds(off, 16)] = jnp.where(pos, y, 0.0)
        invcv[pl.ds(off, 16)] = 1.0 / (d16 + 2.0)
        return 0
    lax.fori_loop(0, NS // 16, dis_loop, 0)
    pltpu.sync_copy(disv.at[pl.ds(s * NS, NS)], deg_sp.at[pl.ds(s * NS, NS)])
    pltpu.sync_copy(invcv.at[pl.ds(s * NS, NS)], invc_sp.at[pl.ds(s * NS, NS)])
    plsc.subcore_barrier()
    pltpu.sync_copy(deg_sp, disv)

    @pl.when(jnp.logical_and(c == 0, s == 0))
    def _():
        pltpu.sync_copy(degv, deg_ref)
        pltpu.sync_copy(invc_sp, invcv)
        pltpu.sync_copy(invcv, invc_ref)
    ns_dis.__exit__(None, None, None)

    with jax.named_scope("sc_lap"):
        def lap_loop(i, _):
            r16 = rowv[pl.ds(i * 16, 16)]
            c16 = colv[pl.ds(i * 16, 16)]
            a = plsc.load_gather(disv, [r16])
            b = plsc.load_gather(disv, [c16])
            lapv[pl.ds(i * 16, 16)] = jnp.where(r16 != c16, -(a * b), 0.0)
            return 0
        lax.fori_loop(0, NV, lap_loop, 0)

    STRIPE = SLAB_FLAT // 16
    ZB = zbuf.shape[0]
    CH = 256
    NCH = EPW // CH
    dumpv = jnp.full((16,), SLAB_FLAT, jnp.int32)
    zerov = jnp.zeros((16,), jnp.float32)
    for slab_i in range(2):
        slab_id = c * 2 + slab_i
        lo = slab_id * SLAB_ROWS
        lo_flat = lo * N

        with jax.named_scope(f"sc_zero{slab_i}"):
            nz = STRIPE // ZB
            zcps = [pltpu.async_copy(zbuf, slab.at[pl.ds(s * STRIPE + k * ZB, ZB)], sem)
                    for k in range(nz)]
            for cp in zcps:
                cp.wait()

        with jax.named_scope(f"sc_compact{slab_i}"):
            def comp_loop(i, ofs):
                r16 = rowv[pl.ds(i * 16, 16)]
                c16 = colv[pl.ds(i * 16, 16)]
                lap16 = lapv[pl.ds(i * 16, 16)]
                flat = r16 * N + c16 - lo_flat
                ok = jnp.logical_and(r16 >= lo, r16 < lo + SLAB_ROWS)
                plsc.store_compressed(cmaj.at[pl.ds(ofs, 16)], flat, mask=ok)
                plsc.store_compressed(clap.at[pl.ds(ofs, 16)], lap16, mask=ok)
                pc = plsc.all_reduce_population_count(ok)
                return ofs + pc[0]
            cnt = lax.fori_loop(0, NV, comp_loop, jnp.int32(0))
            cmaj[pl.ds(cnt, 16)] = dumpv
            clap[pl.ds(cnt, 16)] = zerov

            def pad_loop(k, _):
                cmaj[pl.ds(k * 16, 16)] = dumpv
                clap[pl.ds(k * 16, 16)] = zerov
                return 0
            nch = lax.div(cnt + CH - 1, CH)
            lax.fori_loop(lax.div(cnt + 16, 16), nch * (CH // 16), pad_loop, 0)
            for k in range(NCH):
                @pl.when(k * CH < cnt)
                def _():
                    def cp_loop(i, _2):
                        cmaj2[k, pl.ds(i * 16, 16)] = cmaj[pl.ds(k * CH + i * 16, 16)]
                        return 0
                    lax.fori_loop(0, CH // 16, cp_loop, 0)
            plsc.subcore_barrier()

        with jax.named_scope(f"sc_scatter{slab_i}"):
            for k in range(NCH):
                @pl.when(k * CH < cnt)
                def _():
                    pltpu.sync_copy(clap.at[pl.ds(k * CH, CH)],
                                    slab.at[cmaj2.at[k]], add=True)
            plsc.subcore_barrier()

        with jax.named_scope(f"sc_wb{slab_i}"):
            pltpu.sync_copy(slab.at[pl.ds(s * STRIPE, STRIPE)],
                            m_ref.at[pl.ds(lo_flat + s * STRIPE, STRIPE)])


def _make_build_m(N, E):
    EPW = E // 16
    CH = 256
    SLAB_FLAT = SLAB_ROWS * N
    mesh = plsc.VectorSubcoreMesh(core_axis_name="c", subcore_axis_name="s")
    return pl.kernel(
        _build_m_body,
        out_type=[
            jax.ShapeDtypeStruct((N * N,), jnp.float32),
            jax.ShapeDtypeStruct((N,), jnp.float32),
            jax.ShapeDtypeStruct((N,), jnp.float32),
        ],
        mesh=mesh,
        compiler_params=pltpu.CompilerParams(
            needs_layout_passes=False, use_tc_tiling_on_sc=False),
        scratch_types=[
            pltpu.VMEM((EPW,), jnp.int32),
            pltpu.VMEM((EPW,), jnp.int32),
            pltpu.VMEM((EPW,), jnp.float32),
            pltpu.VMEM((EPW + 16,), jnp.int32),
            pltpu.VMEM((EPW + 16,), jnp.float32),
            pltpu.VMEM((EPW // CH, CH), jnp.int32),
            pltpu.VMEM((N,), jnp.float32),
            pltpu.VMEM((N,), jnp.float32),
            pltpu.VMEM((N,), jnp.float32),
            pltpu.VMEM((8192,), jnp.float32),
            pltpu.VMEM((N // 16,), jnp.float32),
            pltpu.VMEM_SHARED((N,), jnp.float32),
            pltpu.VMEM_SHARED((N,), jnp.float32),
            pltpu.VMEM_SHARED((SLAB_FLAT + 16,), jnp.float32),
            pltpu.SemaphoreType.DMA,
        ],
    )


def _prep_body(sa_ref, x_ref, tax0_ref):
    TB = sa_ref.shape[1]
    rloc = lax.broadcasted_iota(jnp.int32, (TB, TB), 0)
    cloc = lax.broadcasted_iota(jnp.int32, (TB, TB), 1)
    d = jnp.sum(jnp.where(rloc == cloc, sa_ref[0], 0.0), axis=1)
    tax0_ref[0] = d[:, None] * x_ref[0]


def _tax1_body(m_ref, sa_ref, tax0_ref, invc_ref, tax1_ref):
    b = pl.program_id(1)
    N = m_ref.shape[1]
    a = m_ref[...] * sa_ref[0]
    tax0 = tax0_ref[0, pl.ds(b * N, N), :]
    acc = jnp.dot(a, tax0, preferred_element_type=jnp.float32)
    tax1_ref[0] = invc_ref[...] * acc


def _out_body(m_ref, tax1_ref, tax0_ref, invc_ref, w_ref, b_ref, out_ref):
    i = pl.program_id(0)
    b = pl.program_id(1)
    N = m_ref.shape[1]
    tax1_full = tax1_ref[0, pl.ds(b * N, N), :]
    s2 = jnp.dot(m_ref[...], tax1_full, preferred_element_type=jnp.float32)
    tax0_t = tax0_ref[0]
    tax2 = 2.0 * invc_ref[...] * s2 - tax0_t
    tax1_t = tax1_ref[0, pl.ds(b * N + i * TN, TN), :]
    out = jnp.dot(tax0_t, w_ref[0], preferred_element_type=jnp.float32)
    out = out + jnp.dot(tax1_t, w_ref[1], preferred_element_type=jnp.float32)
    out = out + jnp.dot(tax2, w_ref[2], preferred_element_type=jnp.float32)
    out_ref[0] = out + b_ref[...]


@jax.jit
def _run(x, edge_index, spatial_attention, weight, bias):
    B, N, F = x.shape
    E = edge_index.shape[1]

    m_flat, deg, invc1 = _make_build_m(N, E)(edge_index)
    M = m_flat.reshape(N, N)
    invc = invc1.reshape(N, 1)

    TB = 128
    tax0 = pl.pallas_call(
        _prep_body,
        grid=(B, N // TB),
        in_specs=[
            pl.BlockSpec((1, TB, TB), lambda b, i: (b, i, i)),
            pl.BlockSpec((1, TB, F), lambda b, i: (b, i, 0)),
        ],
        out_specs=pl.BlockSpec((1, TB, F), lambda b, i: (b, i, 0)),
        out_shape=jax.ShapeDtypeStruct((B, N, F), jnp.float32),
    )(spatial_attention, x)

    T = N // TN
    grid = (T, B)
    tax0_flat = tax0.reshape(1, B * N, F)
    tax1 = pl.pallas_call(
        _tax1_body,
        grid=grid,
        in_specs=[
            pl.BlockSpec((TN, N), lambda i, b: (i, 0)),
            pl.BlockSpec((1, TN, N), lambda i, b: (b, i, 0)),
            pl.BlockSpec((1, B * N, F), lambda i, b: (0, 0, 0)),
            pl.BlockSpec((TN, 1), lambda i, b: (i, 0)),
        ],
        out_specs=pl.BlockSpec((1, TN, F), lambda i, b: (b, i, 0)),
        out_shape=jax.ShapeDtypeStruct((B, N, F), jnp.float32),
    )(M, spatial_attention, tax0_flat, invc)

    out = pl.pallas_call(
        _out_body,
        grid=grid,
        in_specs=[
            pl.BlockSpec((TN, N), lambda i, b: (i, 0)),
            pl.BlockSpec((1, B * N, F), lambda i, b: (0, 0, 0)),
            pl.BlockSpec((1, TN, F), lambda i, b: (b, i, 0)),
            pl.BlockSpec((TN, 1), lambda i, b: (i, 0)),
            pl.BlockSpec((3, F, F), lambda i, b: (0, 0, 0)),
            pl.BlockSpec((1, F), lambda i, b: (0, 0)),
        ],
        out_specs=pl.BlockSpec((1, TN, F), lambda i, b: (b, i, 0)),
        out_shape=jax.ShapeDtypeStruct((B, N, F), jnp.float32),
    )(M, tax1.reshape(1, B * N, F), tax0, invc, weight, bias.reshape(1, F))
    return out


def kernel(x, edge_index, spatial_attention, weight, bias):
    return _run(x, edge_index, spatial_attention, weight, bias)


# speedup vs baseline: 1.3454x; 1.0010x over previous
import jax
import jax.numpy as jnp
from jax import lax
from jax.experimental import pallas as pl
from jax.experimental.pallas import tpu as pltpu
from jax.experimental.pallas import tpu_sc as plsc

TN = 512
SLAB_ROWS = 512


def _build_m_body(ei_ref, m_ref, deg_ref, invc_ref,
                  rowv, colv, lapv, cmaj, clap, cmaj2, degv, disv, invcv,
                  zbuf, z1d, deg_sp, invc_sp, slab, sem):
    N = deg_ref.shape[0]
    E = ei_ref.shape[1]
    EPW = E // 16
    NV = EPW // 16
    SLAB_FLAT = SLAB_ROWS * N
    c = lax.axis_index("c")
    s = lax.axis_index("s")
    base = s * EPW

    with jax.named_scope("sc_load_edges"):
        pltpu.sync_copy(ei_ref.at[0, pl.ds(base, EPW)], rowv)
        pltpu.sync_copy(ei_ref.at[1, pl.ds(base, EPW)], colv)

        def zloop(i, _):
            zbuf[pl.ds(i * 16, 16)] = jnp.zeros((16,), jnp.float32)
            return 0
        lax.fori_loop(0, zbuf.shape[0] // 16, zloop, 0)

        def z1loop(i, _):
            z1d[pl.ds(i * 16, 16)] = jnp.zeros((16,), jnp.float32)
            return 0
        lax.fori_loop(0, z1d.shape[0] // 16, z1loop, 0)

    with jax.named_scope("sc_deg"):
        pltpu.sync_copy(z1d, deg_sp.at[pl.ds(s * (N // 16), N // 16)])
        plsc.subcore_barrier()

        def ones_loop(i, _):
            r16 = rowv[pl.ds(i * 16, 16)]
            c16 = colv[pl.ds(i * 16, 16)]
            lapv[pl.ds(i * 16, 16)] = jnp.where(r16 != c16, 1.0, 0.0)
            return 0
        lax.fori_loop(0, NV, ones_loop, 0)
        pltpu.sync_copy(lapv, deg_sp.at[rowv], add=True)
        plsc.subcore_barrier()

    ns_dis = jax.named_scope("sc_dis")
    ns_dis.__enter__()
    pltpu.sync_copy(deg_sp, degv)
    plsc.subcore_barrier()
    NS = N // 16

    def dis_loop(i, _):
        off = s * NS + i * 16
        d16 = degv[pl.ds(off, 16)]
        pos = d16 > 0.0
        dsafe = jnp.where(pos, d16, 1.0)
        y = 1.0 / dsafe
        half = 0.5 * dsafe
        for _ in range(18):
            y = y * (1.5 - half * y * y)
        disv[pl.ds(off, 16)] = jnp.where(pos, y, 0.0)
        invcv[pl.ds(off, 16)] = 1.0 / (d16 + 2.0)
        return 0
    lax.fori_loop(0, NS // 16, dis_loop, 0)
    pltpu.sync_copy(disv.at[pl.ds(s * NS, NS)], deg_sp.at[pl.ds(s * NS, NS)])
    pltpu.sync_copy(invcv.at[pl.ds(s * NS, NS)], invc_sp.at[pl.ds(s * NS, NS)])
    plsc.subcore_barrier()
    pltpu.sync_copy(deg_sp, disv)

    @pl.when(jnp.logical_and(c == 0, s == 0))
    def _():
        pltpu.sync_copy(degv, deg_ref)
        pltpu.sync_copy(invc_sp, invcv)
        pltpu.sync_copy(invcv, invc_ref)
    ns_dis.__exit__(None, None, None)

    with jax.named_scope("sc_lap"):
        def lap_loop(i, _):
            r16 = rowv[pl.ds(i * 16, 16)]
            c16 = colv[pl.ds(i * 16, 16)]
            a = plsc.load_gather(disv, [r16])
            b = plsc.load_gather(disv, [c16])
            lapv[pl.ds(i * 16, 16)] = jnp.where(r16 != c16, -(a * b), 0.0)
            return 0
        lax.fori_loop(0, NV, lap_loop, 0)

    STRIPE = SLAB_FLAT // 16
    ZB = zbuf.shape[0]
    CH = 256
    NCH = EPW // CH
    dumpv = jnp.full((16,), SLAB_FLAT, jnp.int32)
    zerov = jnp.zeros((16,), jnp.float32)
    for slab_i in range(2):
        slab_id = c * 2 + slab_i
        lo = slab_id * SLAB_ROWS
        lo_flat = lo * N

        with jax.named_scope(f"sc_zero{slab_i}"):
            nz = STRIPE // ZB
            zcps = [pltpu.async_copy(zbuf, slab.at[pl.ds(s * STRIPE + k * ZB, ZB)], sem)
                    for k in range(nz)]
            for cp in zcps:
                cp.wait()

        with jax.named_scope(f"sc_compact{slab_i}"):
            def comp_loop(i, ofs):
                r16 = rowv[pl.ds(i * 16, 16)]
                c16 = colv[pl.ds(i * 16, 16)]
                lap16 = lapv[pl.ds(i * 16, 16)]
                flat = r16 * N + c16 - lo_flat
                ok = jnp.logical_and(r16 >= lo, r16 < lo + SLAB_ROWS)
                plsc.store_compressed(cmaj.at[pl.ds(ofs, 16)], flat, mask=ok)
                plsc.store_compressed(clap.at[pl.ds(ofs, 16)], lap16, mask=ok)
                pc = plsc.all_reduce_population_count(ok)
                return ofs + pc[0]
            cnt = lax.fori_loop(0, NV, comp_loop, jnp.int32(0))
            cmaj[pl.ds(cnt, 16)] = dumpv
            clap[pl.ds(cnt, 16)] = zerov

            def pad_loop(k, _):
                cmaj[pl.ds(k * 16, 16)] = dumpv
                clap[pl.ds(k * 16, 16)] = zerov
                return 0
            nch = lax.div(cnt + CH - 1, CH)
            lax.fori_loop(lax.div(cnt + 16, 16), nch * (CH // 16), pad_loop, 0)
            for k in range(NCH):
                @pl.when(k * CH < cnt)
                def _():
                    def cp_loop(i, _2):
                        cmaj2[k, pl.ds(i * 16, 16)] = cmaj[pl.ds(k * CH + i * 16, 16)]
                        return 0
                    lax.fori_loop(0, CH // 16, cp_loop, 0)
            plsc.subcore_barrier()

        with jax.named_scope(f"sc_scatter{slab_i}"):
            for k in range(NCH):
                @pl.when(k * CH < cnt)
                def _():
                    pltpu.sync_copy(clap.at[pl.ds(k * CH, CH)],
                                    slab.at[cmaj2.at[k]], add=True)
            plsc.subcore_barrier()

        with jax.named_scope(f"sc_wb{slab_i}"):
            pltpu.sync_copy(slab.at[pl.ds(s * STRIPE, STRIPE)],
                            m_ref.at[pl.ds(lo_flat + s * STRIPE, STRIPE)])


def _make_build_m(N, E):
    EPW = E // 16
    CH = 256
    SLAB_FLAT = SLAB_ROWS * N
    mesh = plsc.VectorSubcoreMesh(core_axis_name="c", subcore_axis_name="s")
    return pl.kernel(
        _build_m_body,
        out_type=[
            jax.ShapeDtypeStruct((N * N,), jnp.float32),
            jax.ShapeDtypeStruct((N,), jnp.float32),
            jax.ShapeDtypeStruct((N,), jnp.float32),
        ],
        mesh=mesh,
        compiler_params=pltpu.CompilerParams(
            needs_layout_passes=False, use_tc_tiling_on_sc=False),
        scratch_types=[
            pltpu.VMEM((EPW,), jnp.int32),
            pltpu.VMEM((EPW,), jnp.int32),
            pltpu.VMEM((EPW,), jnp.float32),
            pltpu.VMEM((EPW + 16,), jnp.int32),
            pltpu.VMEM((EPW + 16,), jnp.float32),
            pltpu.VMEM((EPW // CH, CH), jnp.int32),
            pltpu.VMEM((N,), jnp.float32),
            pltpu.VMEM((N,), jnp.float32),
            pltpu.VMEM((N,), jnp.float32),
            pltpu.VMEM((8192,), jnp.float32),
            pltpu.VMEM((N // 16,), jnp.float32),
            pltpu.VMEM_SHARED((N,), jnp.float32),
            pltpu.VMEM_SHARED((N,), jnp.float32),
            pltpu.VMEM_SHARED((SLAB_FLAT + 16,), jnp.float32),
            pltpu.SemaphoreType.DMA,
        ],
    )


def _prep_body(sa_ref, x_ref, tax0_ref):
    TB = sa_ref.shape[1]
    rloc = lax.broadcasted_iota(jnp.int32, (TB, TB), 0)
    cloc = lax.broadcasted_iota(jnp.int32, (TB, TB), 1)
    d = jnp.sum(jnp.where(rloc == cloc, sa_ref[0], 0.0), axis=1)
    tax0_ref[0] = d[:, None] * x_ref[0]


def _tax1_body(m_ref, sa_ref, tax0_ref, invc_ref, tax1_ref):
    b = pl.program_id(1)
    N = m_ref.shape[1]
    a = m_ref[...] * sa_ref[0]
    tax0 = tax0_ref[0, pl.ds(b * N, N), :]
    acc = jnp.dot(a, tax0, preferred_element_type=jnp.float32)
    tax1_ref[0] = invc_ref[...] * acc


def _out_body(m_ref, tax1_ref, tax0_ref, invc_ref, w_ref, b_ref, out_ref):
    i = pl.program_id(0)
    b = pl.program_id(1)
    N = m_ref.shape[1]
    tax1_full = tax1_ref[0, pl.ds(b * N, N), :]
    s2 = jnp.dot(m_ref[...], tax1_full, preferred_element_type=jnp.float32)
    tax0_t = tax0_ref[0]
    tax2 = 2.0 * invc_ref[...] * s2 - tax0_t
    tax1_t = tax1_ref[0, pl.ds(b * N + i * TN, TN), :]
    out = jnp.dot(tax0_t, w_ref[0], preferred_element_type=jnp.float32)
    out = out + jnp.dot(tax1_t, w_ref[1], preferred_element_type=jnp.float32)
    out = out + jnp.dot(tax2, w_ref[2], preferred_element_type=jnp.float32)
    out_ref[0] = out + b_ref[...]


@jax.jit
def _run(x, edge_index, spatial_attention, weight, bias):
    B, N, F = x.shape
    E = edge_index.shape[1]

    m_flat, deg, invc1 = _make_build_m(N, E)(edge_index)
    M = m_flat.reshape(N, N)
    invc = invc1.reshape(N, 1)

    TB = 128
    tax0 = pl.pallas_call(
        _prep_body,
        grid=(B, N // TB),
        in_specs=[
            pl.BlockSpec((1, TB, TB), lambda b, i: (b, i, i)),
            pl.BlockSpec((1, TB, F), lambda b, i: (b, i, 0)),
        ],
        out_specs=pl.BlockSpec((1, TB, F), lambda b, i: (b, i, 0)),
        out_shape=jax.ShapeDtypeStruct((B, N, F), jnp.float32),
    )(spatial_attention, x)

    T = N // TN
    grid = (T, B)
    tax0_flat = tax0.reshape(1, B * N, F)
    tax1 = pl.pallas_call(
        _tax1_body,
        grid=grid,
        in_specs=[
            pl.BlockSpec((TN, N), lambda i, b: (i, 0)),
            pl.BlockSpec((1, TN, N), lambda i, b: (b, i, 0)),
            pl.BlockSpec((1, B * N, F), lambda i, b: (0, 0, 0)),
            pl.BlockSpec((TN, 1), lambda i, b: (i, 0)),
        ],
        out_specs=pl.BlockSpec((1, TN, F), lambda i, b: (b, i, 0)),
        out_shape=jax.ShapeDtypeStruct((B, N, F), jnp.float32),
    )(M, spatial_attention, tax0_flat, invc)

    out = pl.pallas_call(
        _out_body,
        grid=grid,
        in_specs=[
            pl.BlockSpec((TN, N), lambda i, b: (i, 0)),
            pl.BlockSpec((1, B * N, F), lambda i, b: (0, 0, 0)),
            pl.BlockSpec((1, TN, F), lambda i, b: (b, i, 0)),
            pl.BlockSpec((TN, 1), lambda i, b: (i, 0)),
            pl.BlockSpec((3, F, F), lambda i, b: (0, 0, 0)),
            pl.BlockSpec((1, F), lambda i, b: (0, 0)),
        ],
        out_specs=pl.BlockSpec((1, TN, F), lambda i, b: (b, i, 0)),
        out_shape=jax.ShapeDtypeStruct((B, N, F), jnp.float32),
    )(M, tax1.reshape(1, B * N, F), tax0, invc, weight, bias.reshape(1, F))
    return out


def kernel(x, edge_index, spatial_attention, weight, bias):
    return _run(x, edge_index, spatial_attention, weight, bias)
